# baseline (speedup 1.0000x reference)
import functools
import math

import jax
import jax.numpy as jnp
from jax.experimental import pallas as pl
from jax.experimental.pallas import tpu as pltpu

_NUM_EXPERTS = 16
_TOP_K = 2
_INV_TEMP = 1.0 / 0.7
_INV_SQRT2 = 1.0 / math.sqrt(2.0)
_BLK = 2048


def _router_body(x_ref, w1_ref, b1_ref, w2_ref, b2_ref, out_ref):
    x = x_ref[...]
    h = (jnp.dot(x, w1_ref[...], preferred_element_type=jnp.float32)
         + b1_ref[...].reshape(1, -1))
    h = 0.5 * h * (1.0 + jax.lax.erf(h * _INV_SQRT2))
    logits = (jnp.dot(h, w2_ref[...], preferred_element_type=jnp.float32)
              + b2_ref[...].reshape(1, -1))
    s = logits * _INV_TEMP
    s = s - jnp.max(s, axis=-1, keepdims=True)
    e = jnp.exp(s)
    gates = e / jnp.sum(e, axis=-1, keepdims=True)

    n_exp = gates.shape[-1]
    row_i = jax.lax.broadcasted_iota(jnp.int32, (n_exp, n_exp), 0)
    col_i = jax.lax.broadcasted_iota(jnp.int32, (n_exp, n_exp), 1)
    ut = (row_i < col_i).astype(jnp.float32)
    m1 = jnp.max(gates, axis=-1, keepdims=True)
    is1 = gates == m1
    pre1 = jnp.dot(is1.astype(jnp.float32), ut,
                   preferred_element_type=jnp.float32)
    occ1 = is1 & (pre1 == 0.0)
    g2 = jnp.where(occ1, -1.0, gates)
    m2 = jnp.max(g2, axis=-1, keepdims=True)
    is2 = g2 == m2
    pre2 = jnp.dot(is2.astype(jnp.float32), ut,
                   preferred_element_type=jnp.float32)
    occ2 = is2 & (pre2 == 0.0)
    mask = occ1 | occ2

    gg = jnp.where(mask, gates, 0.0)
    out_ref[...] = gg / (m1 + m2 + 1e-8)


@functools.partial(jax.jit, static_argnames=())
def kernel(instrument_logits, W1, b1, W2, b2):
    n_tokens, in_dim = instrument_logits.shape
    hidden = W1.shape[1]
    n_exp = W2.shape[1]
    blk = min(_BLK, n_tokens)
    grid = (n_tokens // blk,)
    return pl.pallas_call(
        _router_body,
        grid=grid,
        in_specs=[
            pl.BlockSpec((blk, in_dim), lambda i: (i, 0)),
            pl.BlockSpec((in_dim, hidden), lambda i: (0, 0)),
            pl.BlockSpec((hidden,), lambda i: (0,)),
            pl.BlockSpec((hidden, n_exp), lambda i: (0, 0)),
            pl.BlockSpec((n_exp,), lambda i: (0,)),
        ],
        out_specs=pl.BlockSpec((blk, n_exp), lambda i: (i, 0)),
        out_shape=jax.ShapeDtypeStruct((n_tokens, n_exp), jnp.float32),
        compiler_params=pltpu.CompilerParams(
            dimension_semantics=("parallel",),
        ),
    )(instrument_logits, W1, b1, W2, b2)


# speedup vs baseline: 1.1309x; 1.1309x over previous
import functools
import math

import jax
import jax.numpy as jnp
from jax.experimental import pallas as pl
from jax.experimental.pallas import tpu as pltpu

_NUM_EXPERTS = 16
_TOP_K = 2
_INV_TEMP = 1.0 / 0.7
_INV_SQRT2 = 1.0 / math.sqrt(2.0)
_BLK = 2048


def _router_body(x_ref, w1t_ref, b1_ref, w2t_ref, b2_ref, out_ref):
    x = x_ref[...]
    w1t = w1t_ref[...]
    w2t = w2t_ref[...]
    h = (jax.lax.dot_general(
            x, w1t, (((1,), (1,)), ((), ())),
            preferred_element_type=jnp.float32)
         + b1_ref[...].reshape(1, -1))
    h = 0.5 * h * (1.0 + jax.lax.erf(h * _INV_SQRT2))
    logits = (jax.lax.dot_general(
                  h, w2t, (((1,), (1,)), ((), ())),
                  preferred_element_type=jnp.float32)
              + b2_ref[...].reshape(1, -1))
    s = logits * _INV_TEMP
    s = s - jnp.max(s, axis=-1, keepdims=True)
    e = jnp.exp(s)
    gates = e / jnp.sum(e, axis=-1, keepdims=True)

    n_exp = gates.shape[-1]
    row_i = jax.lax.broadcasted_iota(jnp.int32, (n_exp, n_exp), 0)
    col_i = jax.lax.broadcasted_iota(jnp.int32, (n_exp, n_exp), 1)
    ut = (row_i < col_i).astype(jnp.float32)
    m1 = jnp.max(gates, axis=-1, keepdims=True)
    is1 = gates == m1
    pre1 = jnp.dot(is1.astype(jnp.float32), ut,
                   preferred_element_type=jnp.float32)
    occ1 = is1 & (pre1 == 0.0)
    g2 = jnp.where(occ1, -1.0, gates)
    m2 = jnp.max(g2, axis=-1, keepdims=True)
    is2 = g2 == m2
    pre2 = jnp.dot(is2.astype(jnp.float32), ut,
                   preferred_element_type=jnp.float32)
    occ2 = is2 & (pre2 == 0.0)
    mask = occ1 | occ2

    gg = jnp.where(mask, gates, 0.0)
    out_ref[...] = gg / (m1 + m2 + 1e-8)


@functools.partial(jax.jit, static_argnames=())
def kernel(instrument_logits, W1, b1, W2, b2):
    n_tokens, in_dim = instrument_logits.shape
    hidden = W1.shape[1]
    n_exp = W2.shape[1]
    blk = min(_BLK, n_tokens)
    grid = (n_tokens // blk,)
    return pl.pallas_call(
        _router_body,
        grid=grid,
        in_specs=[
            pl.BlockSpec((blk, in_dim), lambda i: (i, 0)),
            pl.BlockSpec((hidden, in_dim), lambda i: (0, 0)),
            pl.BlockSpec((hidden,), lambda i: (0,)),
            pl.BlockSpec((n_exp, hidden), lambda i: (0, 0)),
            pl.BlockSpec((n_exp,), lambda i: (0,)),
        ],
        out_specs=pl.BlockSpec((blk, n_exp), lambda i: (i, 0)),
        out_shape=jax.ShapeDtypeStruct((n_tokens, n_exp), jnp.float32),
        compiler_params=pltpu.CompilerParams(
            dimension_semantics=("parallel",),
        ),
    )(instrument_logits, W1.T, b1, W2.T, b2)


# speedup vs baseline: 1.3364x; 1.1818x over previous
import functools
import math

import jax
import jax.numpy as jnp
from jax.experimental import pallas as pl
from jax.experimental.pallas import tpu as pltpu

_NUM_EXPERTS = 16
_TOP_K = 2
_INV_TEMP = 1.0 / 0.7
_INV_SQRT2 = 1.0 / math.sqrt(2.0)
_BLK = 2048


def _router_body(x_ref, w1t_ref, b1_ref, w2t_ref, b2_ref, out_ref):
    x = x_ref[...]
    w1t = w1t_ref[...]
    w2t = w2t_ref[...]
    h = (jax.lax.dot_general(
            x, w1t, (((1,), (1,)), ((), ())),
            preferred_element_type=jnp.float32)
         + b1_ref[...].reshape(1, -1))
    h = 0.5 * h * (1.0 + jax.lax.erf(h * _INV_SQRT2))
    logits = (jax.lax.dot_general(
                  h, w2t, (((1,), (1,)), ((), ())),
                  preferred_element_type=jnp.float32)
              + b2_ref[...].reshape(1, -1))
    s = logits * _INV_TEMP
    s = s - jnp.max(s, axis=-1, keepdims=True)
    e = jnp.exp(s)
    gates = e / jnp.sum(e, axis=-1, keepdims=True)

    n_exp = gates.shape[-1]
    row_i = jax.lax.broadcasted_iota(jnp.int32, (n_exp, n_exp), 0)
    col_i = jax.lax.broadcasted_iota(jnp.int32, (n_exp, n_exp), 1)
    ut = (row_i < col_i).astype(jnp.float32)
    m1 = jnp.max(gates, axis=-1, keepdims=True)
    is1 = gates == m1
    pre1 = jnp.dot(is1.astype(jnp.float32), ut,
                   preferred_element_type=jnp.float32)
    occ1 = is1 & (pre1 == 0.0)
    g2 = jnp.where(occ1, -1.0, gates)
    m2 = jnp.max(g2, axis=-1, keepdims=True)
    is2 = g2 == m2
    pre2 = jnp.dot(is2.astype(jnp.float32), ut,
                   preferred_element_type=jnp.float32)
    occ2 = is2 & (pre2 == 0.0)
    mask = occ1 | occ2

    gg = jnp.where(mask, gates, 0.0)
    out_ref[...] = (gg / (m1 + m2 + 1e-8)).T


@functools.partial(jax.jit, static_argnames=())
def kernel(instrument_logits, W1, b1, W2, b2):
    n_tokens, in_dim = instrument_logits.shape
    hidden = W1.shape[1]
    n_exp = W2.shape[1]
    blk = min(_BLK, n_tokens)
    grid = (n_tokens // blk,)
    return pl.pallas_call(
        _router_body,
        grid=grid,
        in_specs=[
            pl.BlockSpec((blk, in_dim), lambda i: (i, 0)),
            pl.BlockSpec((hidden, in_dim), lambda i: (0, 0)),
            pl.BlockSpec((hidden,), lambda i: (0,)),
            pl.BlockSpec((n_exp, hidden), lambda i: (0, 0)),
            pl.BlockSpec((n_exp,), lambda i: (0,)),
        ],
        out_specs=pl.BlockSpec((n_exp, blk), lambda i: (0, i)),
        out_shape=jax.ShapeDtypeStruct((n_exp, n_tokens), jnp.float32),
        compiler_params=pltpu.CompilerParams(
            dimension_semantics=("parallel",),
        ),
    )(instrument_logits, W1.T, b1, W2.T, b2).T
